# baseline (device time: 94556 ns/iter reference)
import functools

import jax
import jax.numpy as jnp
from jax import lax
from jax.experimental import pallas as pl
from jax.experimental.pallas import tpu as pltpu

N_DEV = 4


def kernel(x, router_W, route_idx, expert_W):
    del router_W
    n_tok, d_model = x.shape
    e_local, _, d_ff = expert_W.shape
    blk = n_tok // N_DEV
    hw = d_ff // 2

    def body(x_ref, idx_ref, w_ref, out_ref, stage_cw, stage_ccw,
             rs_send_cw, rs_recv_cw, rs_send_ccw, rs_recv_ccw,
             ag_send_cw, ag_recv_cw, ag_send_ccw, ag_recv_ccw):
        my_pos = lax.axis_index("i")
        left = (my_pos - 1) % N_DEV
        right = (my_pos + 1) % N_DEV

        def rows(b):
            return pl.ds((b % N_DEV) * blk, blk)

        cw_cols = pl.ds(0, hw)
        ccw_cols = pl.ds(hw, hw)

        def compute_block(b):
            r = rows(b)
            xb = x_ref[r, :]
            routeb = idx_ref[r, :]
            acc = jnp.zeros((blk, d_ff), jnp.float32)
            for el in range(e_local):
                ge = my_pos * e_local + el
                mask = (routeb == ge).astype(jnp.float32)
                acc = acc + jnp.dot(
                    xb * mask, w_ref[el], preferred_element_type=jnp.float32
                )
            out_ref[r, :] = acc

        def rs_step(s):
            cw = pltpu.make_async_remote_copy(
                src_ref=out_ref.at[rows(my_pos - s), cw_cols],
                dst_ref=stage_cw.at[s],
                send_sem=rs_send_cw.at[s],
                recv_sem=rs_recv_cw.at[s],
                device_id=(right,),
                device_id_type=pl.DeviceIdType.MESH,
            )
            ccw = pltpu.make_async_remote_copy(
                src_ref=out_ref.at[rows(my_pos + s), ccw_cols],
                dst_ref=stage_ccw.at[s],
                send_sem=rs_send_ccw.at[s],
                recv_sem=rs_recv_ccw.at[s],
                device_id=(left,),
                device_id_type=pl.DeviceIdType.MESH,
            )
            cw.start()
            ccw.start()
            return cw, ccw

        def rs_finish(s, cw, ccw):
            cw.wait_recv()
            ccw.wait_recv()
            r_cw = rows(my_pos - 1 - s)
            out_ref[r_cw, cw_cols] = out_ref[r_cw, cw_cols] + stage_cw[s]
            r_ccw = rows(my_pos + 1 + s)
            out_ref[r_ccw, ccw_cols] = out_ref[r_ccw, ccw_cols] + stage_ccw[s]

        compute_block(my_pos)

        barrier_sem = pltpu.get_barrier_semaphore()
        for nbr in [left, right]:
            pl.semaphore_signal(
                barrier_sem, inc=1,
                device_id=(nbr,), device_id_type=pl.DeviceIdType.MESH,
            )
        pl.semaphore_wait(barrier_sem, 2)

        in_flight = []

        cw0, ccw0 = rs_step(0)
        in_flight += [cw0, ccw0]
        compute_block(my_pos - 1)
        compute_block(my_pos + 1)
        rs_finish(0, cw0, ccw0)

        cw1, ccw1 = rs_step(1)
        in_flight += [cw1, ccw1]
        compute_block(my_pos + 2)
        rs_finish(1, cw1, ccw1)

        cw2, ccw2 = rs_step(2)
        in_flight += [cw2, ccw2]
        rs_finish(2, cw2, ccw2)

        for s in range(N_DEV - 1):
            cw = pltpu.make_async_remote_copy(
                src_ref=out_ref.at[rows(my_pos + 1 - s), cw_cols],
                dst_ref=out_ref.at[rows(my_pos + 1 - s), cw_cols],
                send_sem=ag_send_cw.at[s],
                recv_sem=ag_recv_cw.at[s],
                device_id=(right,),
                device_id_type=pl.DeviceIdType.MESH,
            )
            ccw = pltpu.make_async_remote_copy(
                src_ref=out_ref.at[rows(my_pos - 1 + s), ccw_cols],
                dst_ref=out_ref.at[rows(my_pos - 1 + s), ccw_cols],
                send_sem=ag_send_ccw.at[s],
                recv_sem=ag_recv_ccw.at[s],
                device_id=(left,),
                device_id_type=pl.DeviceIdType.MESH,
            )
            cw.start()
            ccw.start()
            in_flight += [cw, ccw]
            cw.wait_recv()
            ccw.wait_recv()

        for d in in_flight:
            d.wait_send()

        @functools.partial(
            pl.run_scoped, second_barrier=pltpu.SemaphoreType.REGULAR
        )
        def _(second_barrier):
            for nbr in [left, right]:
                pl.semaphore_signal(
                    second_barrier, inc=1,
                    device_id=(nbr,), device_id_type=pl.DeviceIdType.MESH,
                )
            pl.semaphore_wait(second_barrier, 2)

    return pl.pallas_call(
        body,
        out_shape=jax.ShapeDtypeStruct((n_tok, d_ff), jnp.float32),
        in_specs=[
            pl.BlockSpec(memory_space=pltpu.VMEM),
            pl.BlockSpec(memory_space=pltpu.VMEM),
            pl.BlockSpec(memory_space=pltpu.VMEM),
        ],
        out_specs=pl.BlockSpec(memory_space=pltpu.VMEM),
        scratch_shapes=[
            pltpu.VMEM((N_DEV - 1, blk, hw), jnp.float32),
            pltpu.VMEM((N_DEV - 1, blk, hw), jnp.float32),
            pltpu.SemaphoreType.DMA((N_DEV - 1,)),
            pltpu.SemaphoreType.DMA((N_DEV - 1,)),
            pltpu.SemaphoreType.DMA((N_DEV - 1,)),
            pltpu.SemaphoreType.DMA((N_DEV - 1,)),
            pltpu.SemaphoreType.DMA((N_DEV - 1,)),
            pltpu.SemaphoreType.DMA((N_DEV - 1,)),
            pltpu.SemaphoreType.DMA((N_DEV - 1,)),
            pltpu.SemaphoreType.DMA((N_DEV - 1,)),
        ],
        compiler_params=pltpu.CompilerParams(collective_id=0),
    )(x, route_idx, expert_W)


# device time: 85376 ns/iter; 1.1075x vs baseline; 1.1075x over previous
import functools

import jax
import jax.numpy as jnp
from jax import lax
from jax.experimental import pallas as pl
from jax.experimental.pallas import tpu as pltpu

N_DEV = 4
N_Q = 2


def kernel(x, router_W, route_idx, expert_W):
    del router_W
    n_tok, d_model = x.shape
    e_local, _, d_ff = expert_W.shape
    blk = n_tok // N_DEV
    hw = d_ff // 2
    qw = hw // N_Q

    def body(x_ref, idx_ref, w_ref, out_ref, stage_cw, stage_ccw,
             rs_send_cw, rs_recv_cw, rs_send_ccw, rs_recv_ccw,
             ag_send_cw, ag_recv_cw, ag_send_ccw, ag_recv_ccw):
        my_pos = lax.axis_index("i")
        left = (my_pos - 1) % N_DEV
        right = (my_pos + 1) % N_DEV

        def rows(b):
            return pl.ds((b % N_DEV) * blk, blk)

        def cols(direction, q):
            return pl.ds(direction * hw + q * qw, qw)

        in_flight = []

        def compute_block(b):
            r = rows(b)
            xb = x_ref[r, :]
            routeb = idx_ref[r, :]
            acc = jnp.zeros((blk, d_ff), jnp.float32)
            for el in range(e_local):
                ge = my_pos * e_local + el
                mask = (routeb == ge).astype(jnp.float32)
                acc = acc + jnp.dot(
                    xb * mask, w_ref[el], preferred_element_type=jnp.float32
                )
            out_ref[r, :] = acc

        def rs_msg(s, q, direction):
            if direction == 0:
                src_b, tgt, stage, ssem, rsem = (
                    my_pos - s, right, stage_cw, rs_send_cw, rs_recv_cw)
            else:
                src_b, tgt, stage, ssem, rsem = (
                    my_pos + s, left, stage_ccw, rs_send_ccw, rs_recv_ccw)
            i = s * N_Q + q
            return pltpu.make_async_remote_copy(
                src_ref=out_ref.at[rows(src_b), cols(direction, q)],
                dst_ref=stage.at[s, q],
                send_sem=ssem.at[i],
                recv_sem=rsem.at[i],
                device_id=(tgt,),
                device_id_type=pl.DeviceIdType.MESH,
            )

        def rs_add(s, q, direction):
            if direction == 0:
                b, stage = my_pos - 1 - s, stage_cw
            else:
                b, stage = my_pos + 1 + s, stage_ccw
            r, c = rows(b), cols(direction, q)
            out_ref[r, c] = out_ref[r, c] + stage[s, q]

        def ag_msg(s, q, direction):
            if direction == 0:
                src_b, tgt, ssem, rsem = (
                    my_pos + 1 - s, right, ag_send_cw, ag_recv_cw)
            else:
                src_b, tgt, ssem, rsem = (
                    my_pos - 1 + s, left, ag_send_ccw, ag_recv_ccw)
            i = s * N_Q + q
            ref = out_ref.at[rows(src_b), cols(direction, q)]
            return pltpu.make_async_remote_copy(
                src_ref=ref,
                dst_ref=ref,
                send_sem=ssem.at[i],
                recv_sem=rsem.at[i],
                device_id=(tgt,),
                device_id_type=pl.DeviceIdType.MESH,
            )

        def start(msgs):
            for m in msgs:
                m.start()
            in_flight.extend(msgs)
            return msgs

        compute_block(my_pos)

        barrier_sem = pltpu.get_barrier_semaphore()
        for nbr in [left, right]:
            pl.semaphore_signal(
                barrier_sem, inc=1,
                device_id=(nbr,), device_id_type=pl.DeviceIdType.MESH,
            )
        pl.semaphore_wait(barrier_sem, 2)

        rs = {}
        ag = {}
        for q in range(N_Q):
            for d in range(2):
                rs[(0, q, d)] = rs_msg(0, q, d)
        start([rs[(0, q, d)] for q in range(N_Q) for d in range(2)])

        compute_block(my_pos - 1)
        compute_block(my_pos + 1)

        for s in range(N_DEV - 1):
            for q in range(N_Q):
                for d in range(2):
                    rs[(s, q, d)].wait_recv()
                    rs_add(s, q, d)
                    if s < N_DEV - 2:
                        nxt = rs_msg(s + 1, q, d)
                        rs[(s + 1, q, d)] = nxt
                        start([nxt])
                    else:
                        a0 = ag_msg(0, q, d)
                        ag[(0, q, d)] = a0
                        start([a0])
            if s == 0:
                compute_block(my_pos + 2)

        for s in range(N_DEV - 1):
            for q in range(N_Q):
                for d in range(2):
                    ag[(s, q, d)].wait_recv()
                    if s < N_DEV - 2:
                        nxt = ag_msg(s + 1, q, d)
                        ag[(s + 1, q, d)] = nxt
                        start([nxt])

        for m in in_flight:
            m.wait_send()

        @functools.partial(
            pl.run_scoped, second_barrier=pltpu.SemaphoreType.REGULAR
        )
        def _(second_barrier):
            for nbr in [left, right]:
                pl.semaphore_signal(
                    second_barrier, inc=1,
                    device_id=(nbr,), device_id_type=pl.DeviceIdType.MESH,
                )
            pl.semaphore_wait(second_barrier, 2)

    n_msgs = (N_DEV - 1) * N_Q
    return pl.pallas_call(
        body,
        out_shape=jax.ShapeDtypeStruct((n_tok, d_ff), jnp.float32),
        in_specs=[
            pl.BlockSpec(memory_space=pltpu.VMEM),
            pl.BlockSpec(memory_space=pltpu.VMEM),
            pl.BlockSpec(memory_space=pltpu.VMEM),
        ],
        out_specs=pl.BlockSpec(memory_space=pltpu.VMEM),
        scratch_shapes=[
            pltpu.VMEM((N_DEV - 1, N_Q, blk, qw), jnp.float32),
            pltpu.VMEM((N_DEV - 1, N_Q, blk, qw), jnp.float32),
            pltpu.SemaphoreType.DMA((n_msgs,)),
            pltpu.SemaphoreType.DMA((n_msgs,)),
            pltpu.SemaphoreType.DMA((n_msgs,)),
            pltpu.SemaphoreType.DMA((n_msgs,)),
            pltpu.SemaphoreType.DMA((n_msgs,)),
            pltpu.SemaphoreType.DMA((n_msgs,)),
            pltpu.SemaphoreType.DMA((n_msgs,)),
            pltpu.SemaphoreType.DMA((n_msgs,)),
        ],
        compiler_params=pltpu.CompilerParams(collective_id=0),
    )(x, route_idx, expert_W)
